# baseline (speedup 1.0000x reference)
import dataclasses
import functools

import jax
import jax.numpy as jnp
from jax import lax
from jax.experimental import pallas as pl
from jax.experimental.pallas import tpu as pltpu
from jax.experimental.pallas import tpu_sc as plsc

BEAM = 8
VOCAB = 1_000_000
SEQ = 128

NTEC = 32
SLAB = VOCAB // 4
CH = 50_000
NCH = SLAB // CH
GROUP = 25
NGRP = CH // (GROUP * 16)
NCAND = 512

TCCHUNK = 65_536
TCN = 16

_NEG = float("-inf")
_POS = float("inf")
_BIGI = 2**30


def _tree(op, xs):
    while len(xs) > 1:
        xs = [op(xs[2 * i], xs[2 * i + 1]) for i in range(len(xs) // 2)] + (
            [xs[-1]] if len(xs) % 2 else [])
    return xs[0]


def _sc_body(d_hbm, cv_hbm, ci_hbm, buf0, buf1, cv_v, ci_v, sem0, sem1):
    cid = lax.axis_index("c")
    sid = lax.axis_index("s")
    wid = cid * 16 + sid
    slab_base = wid * SLAB
    vocab_base = (wid % 4) * SLAB

    bufs = (buf0, buf1)
    sems = (sem0, sem1)

    def copy(c):
        return pltpu.make_async_copy(
            d_hbm.at[pl.ds(slab_base + c * CH, CH)],
            bufs[c % 2], sems[c % 2])

    lanes = lax.iota(jnp.int32, 16)

    copy(0).start()
    maxv = jnp.full((16,), _NEG, jnp.float32)
    for c in range(NCH):
        copy(c).wait()
        if c + 1 < NCH:
            copy(c + 1).start()
        buf = bufs[c % 2]

        def g1(i, mv, buf=buf):
            base = i * (GROUP * 16)
            xs = [buf[pl.ds(base + 16 * k, 16)] for k in range(GROUP)]
            return jnp.maximum(mv, _tree(jnp.maximum, xs))

        maxv = lax.fori_loop(0, NGRP, g1, maxv)

    sk = plsc.sort_key_val(maxv, maxv, descending=True)
    if isinstance(sk, (tuple, list)):
        sk = sk[0]
    t_thresh = jnp.min(jnp.where(lanes < 10, sk, _POS))

    for k in range(NCAND // 16):
        cv_v[pl.ds(16 * k, 16)] = jnp.full((16,), _NEG, jnp.float32)
        ci_v[pl.ds(16 * k, 16)] = jnp.full((16,), _BIGI, jnp.int32)

    copy(0).start()
    cursor = jnp.int32(0)
    for c in range(NCH):
        copy(c).wait()
        if c + 1 < NCH:
            copy(c + 1).start()
        buf = bufs[c % 2]
        chunk_gbase = vocab_base + c * CH

        def g2(i, cur, buf=buf, chunk_gbase=chunk_gbase):
            base = i * (GROUP * 16)
            xs = [buf[pl.ds(base + 16 * k, 16)] for k in range(GROUP)]
            gm = _tree(jnp.maximum, xs)
            anyhit = jnp.max(gm) >= t_thresh

            def do_collect(cur2):
                for k in range(GROUP):
                    x = buf[pl.ds(base + 16 * k, 16)]
                    m = x >= t_thresh
                    cnt = jnp.sum(jnp.where(m, 1, 0))
                    cc = jnp.minimum(cur2, NCAND - 16)
                    plsc.store_compressed(cv_v.at[pl.ds(cc, 16)], x, mask=m)
                    gidx = lanes + (chunk_gbase + base + 16 * k)
                    plsc.store_compressed(ci_v.at[pl.ds(cc, 16)], gidx, mask=m)
                    cur2 = cur2 + cnt
                return cur2

            return lax.cond(anyhit, do_collect, lambda cur2: cur2, cur)

        cursor = lax.fori_loop(0, NGRP, g2, cursor)

    pltpu.sync_copy(cv_v, cv_hbm.at[wid])
    pltpu.sync_copy(ci_v, ci_hbm.at[wid])


_sc_cp = pltpu.CompilerParams()
if "needs_layout_passes" in pltpu.CompilerParams.__dataclass_fields__:
    _sc_cp = dataclasses.replace(_sc_cp, needs_layout_passes=False)

_sc_scan = pl.kernel(
    _sc_body,
    compiler_params=_sc_cp,
    out_type=[
        jax.ShapeDtypeStruct((NTEC, NCAND), jnp.float32),
        jax.ShapeDtypeStruct((NTEC, NCAND), jnp.int32),
    ],
    mesh=plsc.VectorSubcoreMesh(core_axis_name="c", subcore_axis_name="s"),
    scratch_types=[
        pltpu.VMEM((CH,), jnp.float32),
        pltpu.VMEM((CH,), jnp.float32),
        pltpu.VMEM((NCAND,), jnp.float32),
        pltpu.VMEM((NCAND,), jnp.int32),
        pltpu.SemaphoreType.DMA,
        pltpu.SemaphoreType.DMA,
    ],
)


def _min_body(d_ref, o_ref, acc_ref):
    i = pl.program_id(0)
    gcol = lax.broadcasted_iota(jnp.int32, (BEAM, TCCHUNK), 1) + i * TCCHUNK
    dmin = jnp.where(gcol < VOCAB, d_ref[...], _POS)
    minc = jnp.min(dmin, axis=1, keepdims=True)

    @pl.when(i == 0)
    def _():
        acc_ref[...] = jnp.full((BEAM, 1), _POS, jnp.float32)

    acc_ref[...] = jnp.minimum(acc_ref[...], minc)

    @pl.when(i == TCN - 1)
    def _():
        o_ref[...] = acc_ref[...]


def _tc_min(d):
    return pl.pallas_call(
        _min_body,
        grid=(TCN,),
        in_specs=[pl.BlockSpec((BEAM, TCCHUNK), lambda i: (0, i))],
        out_specs=pl.BlockSpec((BEAM, 1), lambda i: (0, 0)),
        out_shape=jax.ShapeDtypeStruct((BEAM, 1), jnp.float32),
        scratch_shapes=[pltpu.VMEM((BEAM, 1), jnp.float32)],
    )(d)


def _extract_topk(vals, idxs, k):
    vs, is_ = [], []
    for _ in range(k):
        m = jnp.max(vals, axis=1, keepdims=True)
        hit = vals == m
        sel = jnp.min(jnp.where(hit, idxs, _BIGI), axis=1, keepdims=True)
        vs.append(m)
        is_.append(sel)
        kill = (idxs == sel) & hit
        vals = jnp.where(kill, _NEG, vals)
    return vs, is_


def _merge_body(cv_ref, ci_ref, minv_ref, gen_ref, scores_ref, step_ref,
                gen_out_ref, sc_out_ref):
    gen = gen_ref[...]
    step = step_ref[0, 0]
    col8 = lax.broadcasted_iota(jnp.int32, (BEAM, SEQ), 1)
    p0 = jnp.sum(jnp.where(col8 == step - 2, gen, 0), axis=1, keepdims=True)
    p1 = jnp.sum(jnp.where(col8 == step - 1, gen, 0), axis=1, keepdims=True)

    cv = cv_ref[...]
    ci = ci_ref[...]
    disq = (ci == p0) | (ci == p1)
    cv = jnp.where(disq, _NEG, cv)
    ci = jnp.where(disq, _BIGI, ci)

    minv = minv_ref[...]
    cv = jnp.concatenate([cv, minv, minv], axis=1)
    ci = jnp.concatenate([ci, p0, p1], axis=1)

    v8, i8 = _extract_topk(cv, ci, BEAM)
    vals8 = jnp.concatenate(v8, axis=1)
    idx8 = jnp.concatenate(i8, axis=1)

    sc = jnp.log(vals8) + scores_ref[...]
    fr = lax.broadcasted_iota(jnp.int32, (BEAM, BEAM), 0)
    fc = lax.broadcasted_iota(jnp.int32, (BEAM, BEAM), 1)
    fi = fr * BEAM + fc

    row8 = lax.broadcasted_iota(jnp.int32, (BEAM, SEQ), 0)
    for j in range(BEAM):
        m = jnp.max(sc)
        sel = jnp.min(jnp.where(sc == m, fi, 64))
        r = sel // BEAM
        tok = jnp.sum(jnp.where(fi == sel, idx8, 0))
        sc_out_ref[j:j + 1, 0:1] = jnp.broadcast_to(m, (1, 1))
        sc = jnp.where(fi == sel, _NEG, sc)

        rowsel = jnp.sum(jnp.where(row8 == r, gen, 0), axis=0, keepdims=True)
        base = gen[j:j + 1, :]
        colv = col8[0:1, :]
        newrow = jnp.where(colv < step, rowsel, base)
        newrow = jnp.where(colv == step, tok, newrow)
        gen_out_ref[j:j + 1, :] = newrow


def _merge(cv, ci, minv, gen_seq, scores, step_arr):
    w = 4 * NCAND
    return pl.pallas_call(
        _merge_body,
        in_specs=[
            pl.BlockSpec((BEAM, w), lambda: (0, 0)),
            pl.BlockSpec((BEAM, w), lambda: (0, 0)),
            pl.BlockSpec((BEAM, 1), lambda: (0, 0)),
            pl.BlockSpec((BEAM, SEQ), lambda: (0, 0)),
            pl.BlockSpec((BEAM, 1), lambda: (0, 0)),
            pl.BlockSpec(memory_space=pltpu.SMEM),
        ],
        out_specs=[
            pl.BlockSpec((BEAM, SEQ), lambda: (0, 0)),
            pl.BlockSpec((BEAM, 1), lambda: (0, 0)),
        ],
        out_shape=[
            jax.ShapeDtypeStruct((BEAM, SEQ), jnp.int32),
            jax.ShapeDtypeStruct((BEAM, 1), jnp.float32),
        ],
    )(cv, ci, minv, gen_seq, scores, step_arr)


@jax.jit
def _run(gen_seq, d, d_flat, scores, step_arr):
    cv, ci = _sc_scan(d_flat)
    minv = _tc_min(d)
    gen_new, sc_new = _merge(
        cv.reshape(BEAM, 4 * NCAND), ci.reshape(BEAM, 4 * NCAND),
        minv, gen_seq, scores, step_arr)
    return gen_new, sc_new


def kernel(gen_seq, dec_output, scores, step):
    d = dec_output.reshape(BEAM, VOCAB)
    d_flat = dec_output.reshape(BEAM * VOCAB)
    scores2 = scores.reshape(BEAM, 1)
    step_arr = jnp.asarray(step, jnp.int32).reshape(1, 1)
    gen_new, sc_new = _run(gen_seq, d, d_flat, scores2, step_arr)
    return gen_new, sc_new.reshape(BEAM)
